# baseline (speedup 1.0000x reference)
import functools

import jax
import jax.numpy as jnp
from jax.experimental import pallas as pl
from jax.experimental.pallas import tpu as pltpu


def _scse_slab(x, wc, wet, bc_row, be_row, ws_row):
    xm = jnp.mean(x, axis=0, keepdims=True)
    z = jax.lax.dot_general(xm, wc, (((1,), (1,)), ((), ())),
                            preferred_element_type=jnp.float32)
    z = jnp.maximum(z + bc_row, 0.0)
    g = jax.lax.dot(z, wet, preferred_element_type=jnp.float32)
    g = jax.nn.sigmoid(g + be_row)
    s = jax.nn.sigmoid(
        jax.lax.dot_general(x, ws_row, (((1,), (1,)), ((), ())),
                            preferred_element_type=jnp.float32))
    return x * (g + s)


def _scse_body(x_hbm, wc_ref, wet_ref, bc_ref, be_ref, ws_ref, o_hbm,
               x_buf, o_buf, in_sem, out_sem, *, n_steps):
    p = pl.program_id(0)
    base = p * n_steps
    cr = wc_ref.shape[0]
    wc = wc_ref[...]
    wet = wet_ref[...]
    bc_row = bc_ref[...].reshape(1, cr)
    be_row = be_ref[...].reshape(1, -1)
    ws_row = ws_ref[...].reshape(1, -1)

    def dma_in(slot, step):
        return pltpu.make_async_copy(x_hbm.at[base + step], x_buf.at[slot],
                                     in_sem.at[slot])

    def dma_out(slot, step):
        return pltpu.make_async_copy(o_buf.at[slot], o_hbm.at[base + step],
                                     out_sem.at[slot])

    dma_in(0, 0).start()

    def body(step, _):
        cur = jax.lax.rem(step, 2)
        nxt = jax.lax.rem(step + 1, 2)

        @pl.when(step + 1 < n_steps)
        def _():
            dma_in(nxt, step + 1).start()

        dma_in(cur, step).wait()

        @pl.when(step >= 2)
        def _():
            dma_out(cur, step - 2).wait()

        o_buf[cur] = _scse_slab(x_buf[cur], wc, wet, bc_row, be_row, ws_row)
        dma_out(cur, step).start()
        return ()

    jax.lax.fori_loop(0, n_steps, body, ())
    dma_out(jax.lax.rem(n_steps - 2, 2), n_steps - 2).wait()
    dma_out(jax.lax.rem(n_steps - 1, 2), n_steps - 1).wait()


def kernel(x_nchw, wc, bc, we, be, ws):
    B, C, H, W = x_nchw.shape
    HW = H * W
    Cr = wc.shape[0]
    n_cores = 2
    n_steps = B // n_cores

    x = jnp.transpose(x_nchw, (0, 2, 3, 1)).reshape(B, HW, C)
    wet = we.T

    out = pl.pallas_call(
        functools.partial(_scse_body, n_steps=n_steps),
        out_shape=jax.ShapeDtypeStruct((B, HW, C), x_nchw.dtype),
        grid=(n_cores,),
        in_specs=[
            pl.BlockSpec(memory_space=pltpu.MemorySpace.HBM),
            pl.BlockSpec((Cr, C), lambda p: (0, 0)),
            pl.BlockSpec((Cr, C), lambda p: (0, 0)),
            pl.BlockSpec((Cr,), lambda p: (0,)),
            pl.BlockSpec((C,), lambda p: (0,)),
            pl.BlockSpec((C,), lambda p: (0,)),
        ],
        out_specs=pl.BlockSpec(memory_space=pltpu.MemorySpace.HBM),
        scratch_shapes=[
            pltpu.VMEM((2, HW, C), jnp.float32),
            pltpu.VMEM((2, HW, C), jnp.float32),
            pltpu.SemaphoreType.DMA((2,)),
            pltpu.SemaphoreType.DMA((2,)),
        ],
        compiler_params=pltpu.CompilerParams(
            dimension_semantics=("parallel",),
            vmem_limit_bytes=48 * 1024 * 1024),
    )(x, wc, wet, bc, be, ws)
    return out.reshape(B, H, W, C).transpose(0, 3, 1, 2)


# speedup vs baseline: 1.1271x; 1.1271x over previous
import jax
import jax.numpy as jnp
from jax.experimental import pallas as pl
from jax.experimental.pallas import tpu as pltpu

_BPB = 2


def _scse_body(x_ref, wc_ref, wet_ref, bc_ref, be_ref, ws_ref, o_ref):
    x = x_ref[...]
    cr = wc_ref.shape[0]

    xm = jnp.mean(x, axis=1)
    z = jax.lax.dot_general(xm, wc_ref[...], (((1,), (1,)), ((), ())),
                            preferred_element_type=jnp.float32)
    z = jnp.maximum(z + bc_ref[...].reshape(1, cr), 0.0)
    g = jax.lax.dot(z, wet_ref[...],
                    preferred_element_type=jnp.float32)
    g = jax.nn.sigmoid(g + be_ref[...].reshape(1, -1))

    bpb, hw, c = x.shape
    s = jax.nn.sigmoid(
        jax.lax.dot_general(x.reshape(bpb * hw, c), ws_ref[...].reshape(1, c),
                            (((1,), (1,)), ((), ())),
                            preferred_element_type=jnp.float32))

    o_ref[...] = x * (g[:, None, :] + s.reshape(bpb, hw, 1))


def kernel(x_nchw, wc, bc, we, be, ws):
    B, C, H, W = x_nchw.shape
    HW = H * W
    Cr = wc.shape[0]

    x = jnp.transpose(x_nchw, (0, 2, 3, 1)).reshape(B, HW, C)
    wet = we.T

    out = pl.pallas_call(
        _scse_body,
        out_shape=jax.ShapeDtypeStruct((B, HW, C), x_nchw.dtype),
        grid=(B // _BPB,),
        in_specs=[
            pl.BlockSpec((_BPB, HW, C), lambda b: (b, 0, 0)),
            pl.BlockSpec((Cr, C), lambda b: (0, 0)),
            pl.BlockSpec((Cr, C), lambda b: (0, 0)),
            pl.BlockSpec((Cr,), lambda b: (0,)),
            pl.BlockSpec((C,), lambda b: (0,)),
            pl.BlockSpec((C,), lambda b: (0,)),
        ],
        out_specs=pl.BlockSpec((_BPB, HW, C), lambda b: (b, 0, 0)),
        compiler_params=pltpu.CompilerParams(
            dimension_semantics=("parallel",),
            vmem_limit_bytes=48 * 1024 * 1024),
    )(x, wc, wet, bc, be, ws)
    return out.reshape(B, H, W, C).transpose(0, 3, 1, 2)
